# baseline (speedup 1.0000x reference)
import functools

import jax
import jax.numpy as jnp
from jax import lax
from jax.experimental import pallas as pl
from jax.experimental.pallas import tpu as pltpu
from jax.experimental.pallas import tpu_sc as plsc

Z_DIM = 64


def _lookup_body(z_hbm, idx_hbm, out_hbm, idx_v, row_v):
    is_worker = (lax.axis_index("c") == 0) & (lax.axis_index("s") == 0)

    @pl.when(is_worker)
    def _():
        pltpu.sync_copy(idx_hbm, idx_v)
        r = idx_v[...][0]
        pltpu.sync_copy(z_hbm.at[pl.ds(r, 1), :], row_v)
        pltpu.sync_copy(row_v, out_hbm)


_lookup = functools.partial(
    pl.kernel,
    out_type=jax.ShapeDtypeStruct((1, Z_DIM), jnp.float32),
    mesh=plsc.VectorSubcoreMesh(core_axis_name="c", subcore_axis_name="s"),
    scratch_types=[
        pltpu.VMEM((16,), jnp.int32),
        pltpu.VMEM((1, Z_DIM), jnp.float32),
    ],
)(_lookup_body)


def kernel(Z, list_index):
    idx16 = jnp.full((16,), list_index, jnp.int32)
    return _lookup(Z, idx16).reshape((Z_DIM,))


# speedup vs baseline: 1.0631x; 1.0631x over previous
import functools

import jax
import jax.numpy as jnp
from jax import lax
from jax.experimental import pallas as pl
from jax.experimental.pallas import tpu as pltpu
from jax.experimental.pallas import tpu_sc as plsc

Z_DIM = 64


def _lookup_body(z_hbm, idx_hbm, out_hbm, idx_s, row_v):
    @pl.when(lax.axis_index("c") == 0)
    def _():
        pltpu.sync_copy(idx_hbm, idx_s)
        r = idx_s[0]
        pltpu.sync_copy(z_hbm.at[pl.ds(r, 1), :], row_v)
        pltpu.sync_copy(row_v, out_hbm)


_lookup = functools.partial(
    pl.kernel,
    out_type=jax.ShapeDtypeStruct((1, Z_DIM), jnp.float32),
    mesh=plsc.ScalarSubcoreMesh(axis_name="c", num_cores=1),
    scratch_types=[
        pltpu.SMEM((16,), jnp.int32),
        pltpu.VMEM_SHARED((1, Z_DIM), jnp.float32),
    ],
)(_lookup_body)


def kernel(Z, list_index):
    idx16 = jnp.full((16,), list_index, jnp.int32)
    return _lookup(Z, idx16).reshape((Z_DIM,))


# speedup vs baseline: 1.1079x; 1.0422x over previous
import functools

import jax
import jax.numpy as jnp
from jax import lax
from jax.experimental import pallas as pl
from jax.experimental.pallas import tpu as pltpu
from jax.experimental.pallas import tpu_sc as plsc

Z_DIM = 64


def _lookup_body(z_hbm, idx_hbm, out_hbm, idx_s):
    @pl.when(lax.axis_index("c") == 0)
    def _():
        pltpu.sync_copy(idx_hbm, idx_s)
        r = idx_s[0]
        pltpu.sync_copy(z_hbm.at[r], out_hbm)


_lookup = functools.partial(
    pl.kernel,
    out_type=jax.ShapeDtypeStruct((Z_DIM,), jnp.float32),
    mesh=plsc.ScalarSubcoreMesh(axis_name="c", num_cores=1),
    scratch_types=[
        pltpu.SMEM((1,), jnp.int32),
    ],
)(_lookup_body)


def kernel(Z, list_index):
    idx = jnp.asarray(list_index, jnp.int32).reshape((1,))
    return _lookup(Z, idx)


# speedup vs baseline: 1.1149x; 1.0063x over previous
import functools

import jax
import jax.numpy as jnp
from jax import lax
from jax.experimental import pallas as pl
from jax.experimental.pallas import tpu as pltpu
from jax.experimental.pallas import tpu_sc as plsc

Z_DIM = 64


def _lookup_body(z_hbm, idx_hbm, out_hbm, idx_s):
    pltpu.sync_copy(idx_hbm, idx_s)
    r = idx_s[0]
    pltpu.sync_copy(z_hbm.at[r], out_hbm)


_lookup = functools.partial(
    pl.kernel,
    out_type=jax.ShapeDtypeStruct((Z_DIM,), jnp.float32),
    mesh=plsc.ScalarSubcoreMesh(axis_name="c", num_cores=1),
    scratch_types=[
        pltpu.SMEM((1,), jnp.int32),
    ],
)(_lookup_body)


def kernel(Z, list_index):
    idx = jnp.asarray(list_index, jnp.int32).reshape((1,))
    return _lookup(Z, idx)
